# baseline (speedup 1.0000x reference)
import jax
import jax.numpy as jnp
from jax import lax
from jax.experimental import pallas as pl

N = 10000
E = 320000
D = 128
BN_EPS = 1e-5


def _mlp_body(x_ref, agg_ref, w1_ref, b1_ref, gamma_ref, beta_ref,
              w2_ref, b2_ref, out_ref):
    dn = (((1,), (1,)), ((), ()))
    h = x_ref[...] + agg_ref[...]
    h1 = lax.dot_general(h, w1_ref[...], dn,
                         preferred_element_type=jnp.float32)
    h1 = h1 + b1_ref[...]
    mean = jnp.mean(h1, axis=0, keepdims=True)
    cent = h1 - mean
    var = jnp.mean(cent * cent, axis=0, keepdims=True)
    hn = gamma_ref[...] * cent / jnp.sqrt(var + BN_EPS) + beta_ref[...]
    hr = jnp.maximum(hn, 0.0)
    h2 = lax.dot_general(hr, w2_ref[...], dn,
                         preferred_element_type=jnp.float32)
    h2 = h2 + b2_ref[...]
    out_ref[...] = jnp.maximum(h2, 0.0)


def kernel(x, edge_index, W1, b1, gamma, beta, W2, b2):
    agg = jax.ops.segment_sum(jnp.take(x, edge_index[0], axis=0),
                              edge_index[1], num_segments=N)
    return pl.pallas_call(
        _mlp_body,
        out_shape=jax.ShapeDtypeStruct((N, D), jnp.float32),
    )(x, agg, W1, b1.reshape(1, D), gamma.reshape(1, D),
      beta.reshape(1, D), W2, b2.reshape(1, D))
